# baseline (speedup 1.0000x reference)
import jax
import jax.numpy as jnp
from jax.experimental import pallas as pl
from jax.experimental.pallas import tpu as pltpu

_IN = 1024
_OUT = 32768
_K = 256
_B = 8
_BLOCK_N = 2048
_NBLK = _OUT // _BLOCK_N
_SUB = 512
_SUBRANK = 4
_JBITS = 16


def _float_key(t):
    i = jax.lax.bitcast_convert_type(t, jnp.int32)
    return jnp.where(i >= 0, i, i ^ jnp.int32(0x7FFFFFFF))


def _key_float(k):
    i = jnp.where(k >= 0, k, k ^ jnp.int32(0x7FFFFFFF))
    return jax.lax.bitcast_convert_type(i, jnp.float32)


def _count_ge(key, cand):
    m = (key >= cand).astype(jnp.int32)
    n = m.shape[1]
    if n < 1024:
        return jnp.sum(m, axis=1, keepdims=True)
    p = [jnp.sum(m[:, i * (n // 8):(i + 1) * (n // 8)], axis=1,
                 keepdims=True) for i in range(8)]
    return ((p[0] + p[1]) + (p[2] + p[3])) + ((p[4] + p[5]) + (p[6] + p[7]))


def _kth_largest_full(key, k):
    msb = jnp.int32(-2147483648)
    prefix_u = jnp.zeros((key.shape[0], 1), jnp.int32)
    for bit in range(31, -1, -1):
        bitval = (1 << bit) if bit < 31 else -(1 << 31)
        cand_u = prefix_u | jnp.int32(bitval)
        cnt = _count_ge(key, cand_u ^ msb)
        prefix_u = jnp.where(cnt >= k, cand_u, prefix_u)
    return prefix_u ^ msb


def _kth_largest_est(key, k, lo_bit):
    msb = jnp.int32(-2147483648)
    prefix_u = jnp.zeros((key.shape[0], 1), jnp.int32)
    for bit in range(31, lo_bit - 1, -1):
        bitval = (1 << bit) if bit < 31 else -(1 << 31)
        cand_u = prefix_u | jnp.int32(bitval)
        cnt = _count_ge(key, cand_u ^ msb)
        prefix_u = jnp.where(cnt >= k, cand_u, prefix_u)
    return prefix_u ^ msb


def _wta_kernel(x_ref, w_ref, b_ref, out_ref, key_ref, th_ref):
    step = pl.program_id(0)
    t_blk = jax.lax.dot_general(
        x_ref[...], w_ref[...],
        (((1,), (1,)), ((), ())),
        preferred_element_type=jnp.float32,
    ) + b_ref[...]
    key_ref[:, pl.ds(step * _BLOCK_N, _BLOCK_N)] = _float_key(t_blk)

    @pl.when(step == _NBLK - 1)
    def _():
        key = key_ref[...]
        est = _kth_largest_est(key[:, :_SUB], _SUBRANK, 14)
        base = (est & jnp.int32(-(1 << 14))) - jnp.int32(1 << _JBITS)
        c_lo = _count_ge(key, base)
        c_hi = _count_ge(key, base + jnp.int32(1 << (_JBITS + 1)))
        good = jnp.all((c_lo >= _K) & (c_hi < _K))

        @pl.when(good)
        def _refine():
            prefix = base
            for bit in range(_JBITS, -1, -1):
                cand = prefix + jnp.int32(1 << bit)
                cnt = _count_ge(key, cand)
                prefix = jnp.where(cnt >= _K, cand, prefix)
            th_ref[...] = jnp.broadcast_to(prefix, (_B, 128))

        @pl.when(jnp.logical_not(good))
        def _full():
            th_ref[...] = jnp.broadcast_to(_kth_largest_full(key, _K),
                                           (_B, 128))

        thresh = th_ref[:, :1]
        kmin = jnp.int32(-2147483648)
        zero = jnp.float32(0.0)

        nge_parts = []
        for c in range(_NBLK):
            kc = key[:, c * _BLOCK_N:(c + 1) * _BLOCK_N]
            ge = kc >= thresh
            nge_parts.append(jnp.sum(ge.astype(jnp.int32), axis=1,
                                     keepdims=True))
            pooled = jnp.max(jnp.where(ge, kc, kmin), axis=0, keepdims=True)
            out_ref[:, c * _BLOCK_N:(c + 1) * _BLOCK_N] = jnp.where(
                pooled == kmin, zero, _key_float(pooled))
        n_ge = sum(nge_parts)

        @pl.when(jnp.any(n_ge > _K))
        def _tie():
            gt = key > thresh
            eq = key == thresh
            r = _K - (n_ge - jnp.sum(eq.astype(jnp.int32), axis=1,
                                     keepdims=True))
            col = jax.lax.broadcasted_iota(jnp.int32, (_B, _OUT), 1)
            mpref = jnp.zeros((_B, 1), jnp.int32)
            for bit in range(15, -1, -1):
                cand = mpref | jnp.int32(1 << bit)
                cntc = jnp.sum((eq & (col < cand)).astype(jnp.int32),
                               axis=1, keepdims=True)
                mpref = jnp.where(cntc <= r, cand, mpref)
            accept = gt | (eq & (col < mpref))
            pooled = jnp.max(jnp.where(accept, key, kmin), axis=0,
                             keepdims=True)
            out_ref[...] = jnp.where(pooled == kmin, zero,
                                     _key_float(pooled))


def kernel(inputs, W, b):
    out = pl.pallas_call(
        _wta_kernel,
        grid=(_NBLK,),
        in_specs=[
            pl.BlockSpec((_B, _IN), lambda i: (0, 0)),
            pl.BlockSpec((_BLOCK_N, _IN), lambda i: (i, 0)),
            pl.BlockSpec((1, _BLOCK_N), lambda i: (0, i)),
        ],
        out_specs=pl.BlockSpec((1, _OUT), lambda i: (0, 0)),
        out_shape=jax.ShapeDtypeStruct((1, _OUT), jnp.float32),
        scratch_shapes=[
            pltpu.VMEM((_B, _OUT), jnp.int32),
            pltpu.VMEM((_B, 128), jnp.int32),
        ],
    )(inputs, W, b.reshape(1, _OUT))
    return out.reshape(_OUT)
